# baseline (speedup 1.0000x reference)
import functools
import jax
import jax.numpy as jnp
from jax import lax
from jax.experimental import pallas as pl
from jax.experimental.pallas import tpu as pltpu
from jax.experimental.pallas import tpu_sc as plsc

NC = 2
NS = 16
LANES = 16
NW = NC * NS


def _sc_pack_tables(embT, c0T, c1T, c2T, emb_tail, c0_tail, c1_tail,
                    c2_tail):
  D, V = embT.shape
  CV = c0T.shape[1]
  emb_cols = V // 128
  cat_cols = CV // 128
  emb_words = V * D
  cat_words = CV * D

  mesh = plsc.VectorSubcoreMesh(
      core_axis_name="c", subcore_axis_name="s", num_cores=NC,
      num_subcores=NS)

  @functools.partial(
      pl.kernel,
      compiler_params=pltpu.CompilerParams(
          use_tc_tiling_on_sc=True, needs_layout_passes=False),
      out_type=(
          jax.ShapeDtypeStruct((emb_words,), jnp.float32),
          jax.ShapeDtypeStruct((cat_words,), jnp.float32),
          jax.ShapeDtypeStruct((cat_words,), jnp.float32),
          jax.ShapeDtypeStruct((cat_words,), jnp.float32),
      ),
      mesh=mesh,
      scratch_types=[
          pltpu.VMEM((32, 128), jnp.float32),
          pltpu.VMEM((4096,), jnp.float32),
          pltpu.VMEM((2048,), jnp.float32),
          pltpu.SemaphoreType.DMA,
      ],
  )
  def a_fn(embT_ref, c0T_ref, c1T_ref, c2T_ref, et_ref, ct0_ref, ct1_ref,
           ct2_ref, oe_ref, oc0_ref, oc1_ref, oc2_ref,
           tbuf, pbuf, tailb, sem):
    wid = lax.axis_index("s") * NC + lax.axis_index("c")
    iota32 = lax.iota(jnp.int32, LANES) * 32

    def pack_table(src_ref, out_ref, ncols):
      nunits = (ncols - wid + NW - 1) // NW

      def unit_body(n, carry):
        c = wid + n * NW
        for d_hi in range(4):
          pltpu.sync_copy(
              src_ref.at[pl.ds(8 * d_hi, 8), pl.ds(c * 128, 128)],
              tbuf.at[pl.ds(8 * d_hi, 8), :])

        def d_body(d, dc):
          for g in range(8):
            v = tbuf[d, pl.ds(16 * g, 16)]
            idx = iota32 + (16 * g * 32 + d)
            plsc.store_scatter(pbuf, [idx], v)
          return dc

        lax.fori_loop(0, 32, d_body, 0)
        pltpu.sync_copy(pbuf, out_ref.at[pl.ds(c * 4096, 4096)])
        return carry

      lax.fori_loop(0, nunits, unit_body, 0)

    pack_table(embT_ref, oe_ref, emb_cols)
    pack_table(c0T_ref, oc0_ref, cat_cols)
    pack_table(c1T_ref, oc1_ref, cat_cols)
    pack_table(c2T_ref, oc2_ref, cat_cols)

    @pl.when(wid == 0)
    def _():
      pltpu.sync_copy(et_ref, tailb)
      pltpu.sync_copy(tailb, oe_ref.at[pl.ds(emb_cols * 4096, 2048)])

    for i, (t_ref, o_ref) in enumerate(
        ((ct0_ref, oc0_ref), (ct1_ref, oc1_ref), (ct2_ref, oc2_ref))):
      @pl.when(wid == i + 1)
      def _(t_ref=t_ref, o_ref=o_ref):
        pltpu.sync_copy(t_ref, tailb.at[pl.ds(0, 1024)])
        pltpu.sync_copy(tailb.at[pl.ds(0, 1024)],
                        o_ref.at[pl.ds(cat_cols * 4096, 1024)])

  return a_fn(embT, c0T, c1T, c2T, emb_tail, c0_tail, c1_tail, c2_tail)


def _sc_gather_pool(encf, add0, add1, add2, tab, cat0, cat1, cat2, B, L):
  D = 32
  rows_per_w = B // NW
  groups = rows_per_w // LANES

  mesh = plsc.VectorSubcoreMesh(
      core_axis_name="c", subcore_axis_name="s", num_cores=NC,
      num_subcores=NS)

  @functools.partial(
      pl.kernel,
      compiler_params=pltpu.CompilerParams(
          use_tc_tiling_on_sc=False, needs_layout_passes=False),
      out_type=(
          jax.ShapeDtypeStruct((B, D), jnp.float32),
          jax.ShapeDtypeStruct((B,), jnp.float32),
          jax.ShapeDtypeStruct((B, D), jnp.float32),
      ),
      mesh=mesh,
      scratch_types=[
          pltpu.VMEM((LANES * 50,), jnp.int32),
          pltpu.VMEM((LANES, 50), jnp.int32),
          pltpu.VMEM((LANES * 50,), jnp.int32),
          pltpu.VMEM((LANES * 50, 128), jnp.float32),
          pltpu.VMEM((3, LANES), jnp.int32),
          pltpu.VMEM((3, LANES), jnp.int32),
          pltpu.VMEM((LANES, 128), jnp.float32),
          pltpu.VMEM((LANES, 128), jnp.float32),
          pltpu.VMEM((LANES, 128), jnp.float32),
          pltpu.VMEM((D, LANES), jnp.float32),
          pltpu.VMEM((LANES,), jnp.float32),
          pltpu.VMEM((LANES, D), jnp.float32),
          pltpu.VMEM((LANES, D), jnp.float32),
          pltpu.SemaphoreType.DMA,
          pltpu.SemaphoreType.DMA,
      ],
  )
  def b_fn(encf_ref, a0_ref, a1_ref, a2_ref, tab_ref, c0_ref, c1_ref,
           c2_ref, pooled_out, cnt_out, cat_out,
           tok_v, gidx_v, colb_v, rows_v, ctok_v, cgidx_v, crows0_v,
           crows1_v, crows2_v, pooled_v, cnt_v, outp_v, outc_v, sem, sem2):
    wid = lax.axis_index("s") * NC + lax.axis_index("c")
    lanes_iota = lax.iota(jnp.int32, LANES)
    zeros16 = jnp.zeros((LANES,), jnp.float32)
    base_rows = lanes_iota * L
    cat_refs = (c0_ref, c1_ref, c2_ref)

    def group_body(g, carry):
      b0 = (wid * groups + g) * LANES
      pltpu.sync_copy(encf_ref.at[pl.ds(b0 * L, LANES * L)], tok_v)
      for i, a_ref in enumerate((a0_ref, a1_ref, a2_ref)):
        pltpu.sync_copy(a_ref.at[pl.ds(b0, LANES)], ctok_v.at[i])
      for k in range(L):
        v = tok_v[pl.ds(16 * k, 16)]
        p = lanes_iota + 16 * k
        pr = p // L
        pc = p - pr * L
        plsc.store_scatter(gidx_v, [pr, pc], lax.shift_right_logical(v, 2))
        colb_v[pl.ds(16 * k, 16)] = lax.shift_left(
            jnp.bitwise_and(v, 3), 5)
      for i in range(3):
        cv = ctok_v[i]
        cgidx_v[i] = lax.shift_right_logical(cv, 2)
        ctok_v[i] = lax.shift_left(jnp.bitwise_and(cv, 3), 5)
      copies = [
          pltpu.async_copy(tab_ref.at[gidx_v.at[j]],
                           rows_v.at[pl.ds(j * L, L), :], sem)
          for j in range(LANES)
      ]
      crows_refs = (crows0_v, crows1_v, crows2_v)
      cat_copies = [
          pltpu.async_copy(cat_refs[i].at[cgidx_v.at[i]], crows_refs[i],
                           sem2)
          for i in range(3)
      ]
      for d in range(D):
        pooled_v[d] = zeros16
      cnt_v[...] = zeros16
      for c in copies:
        c.wait()

      def token_body(t, tc):
        ridx = base_rows + t
        colv = plsc.load_gather(colb_v, [ridx])
        s0 = s1 = s2 = s3 = zeros16
        for d in range(D):
          v = plsc.load_gather(rows_v, [ridx, colv + d])
          plsc.addupdate(pooled_v.at[d], v)
          if d % 4 == 0:
            s0 = s0 + v
          elif d % 4 == 1:
            s1 = s1 + v
          elif d % 4 == 2:
            s2 = s2 + v
          else:
            s3 = s3 + v
        s = (s0 + s1) + (s2 + s3)
        plsc.addupdate(
            cnt_v.at[:],
            jnp.where(s != 0.0, jnp.float32(1.0), jnp.float32(0.0)))
        return tc

      lax.fori_loop(0, L, token_body, 0)

      for c in cat_copies:
        c.wait()
      cb0 = ctok_v[0]
      cb1 = ctok_v[1]
      cb2 = ctok_v[2]
      for d in range(D):
        dsplat = jnp.full((LANES,), d, jnp.int32)
        vc = plsc.load_gather(crows0_v, [lanes_iota, cb0 + d])
        vc = vc + plsc.load_gather(crows1_v, [lanes_iota, cb1 + d])
        vc = vc + plsc.load_gather(crows2_v, [lanes_iota, cb2 + d])
        plsc.store_scatter(outc_v, [lanes_iota, dsplat], vc)
        plsc.store_scatter(outp_v, [lanes_iota, dsplat], pooled_v[d])
      pltpu.sync_copy(outp_v, pooled_out.at[pl.ds(b0, LANES), :])
      pltpu.sync_copy(outc_v, cat_out.at[pl.ds(b0, LANES), :])
      pltpu.sync_copy(cnt_v, cnt_out.at[pl.ds(b0, LANES)])
      return carry

    lax.fori_loop(0, groups, group_body, 0)

  return b_fn(encf, add0, add1, add2, tab, cat0, cat1, cat2)


def _tc_finalize_matmul(pooled, cnt, cat_sum, fc_w, fc_b):
  B, D = pooled.shape
  C = fc_w.shape[1]
  BB = 512
  cnt2 = cnt.reshape(B, 1)
  fb2 = fc_b.reshape(1, C)

  def tc_body(p_ref, c_ref, cat_ref, w_ref, b_ref, o_ref):
    q = p_ref[...] / c_ref[...]
    q = jnp.where(q != q, jnp.float32(0.0), q)
    big = jnp.float32(3.4028234663852886e38)
    q = jnp.where(q == jnp.inf, big, q)
    q = jnp.where(q == -jnp.inf, -big, q)
    x = q + cat_ref[...]
    o_ref[...] = (
        jnp.dot(x, w_ref[...], preferred_element_type=jnp.float32)
        + b_ref[...])

  return pl.pallas_call(
      tc_body,
      grid=(B // BB,),
      in_specs=[
          pl.BlockSpec((BB, D), lambda i: (i, 0)),
          pl.BlockSpec((BB, 1), lambda i: (i, 0)),
          pl.BlockSpec((BB, D), lambda i: (i, 0)),
          pl.BlockSpec((D, C), lambda i: (0, 0)),
          pl.BlockSpec((1, C), lambda i: (0, 0)),
      ],
      out_specs=pl.BlockSpec((BB, C), lambda i: (i, 0)),
      out_shape=jax.ShapeDtypeStruct((B, C), jnp.float32),
  )(pooled, cnt2, cat_sum, fc_w, fb2)


def kernel(encoded_text, additional_inputs, emb_table, cat_emb0, cat_emb1,
           cat_emb2, fc_w, fc_b):
  B, L = encoded_text.shape
  V, D = emb_table.shape
  CV = cat_emb0.shape[0]
  emb_main = (V // 128) * 128
  cat_main = (CV // 128) * 128

  enc = encoded_text.astype(jnp.int32)
  add_i = additional_inputs.astype(jnp.int32)

  emb_tail = emb_table[emb_main:, :].reshape(-1)
  c_tails = [t[cat_main:, :].reshape(-1)
             for t in (cat_emb0, cat_emb1, cat_emb2)]

  oe, oc0, oc1, oc2 = _sc_pack_tables(
      emb_table.T, cat_emb0.T, cat_emb1.T, cat_emb2.T,
      emb_tail, *c_tails)

  pooled, cnt, cat_sum = _sc_gather_pool(
      enc.reshape(-1), add_i[:, 0], add_i[:, 1], add_i[:, 2],
      oe.reshape(-1, 128), oc0.reshape(-1, 128), oc1.reshape(-1, 128),
      oc2.reshape(-1, 128), B, L)
  return _tc_finalize_matmul(pooled, cnt, cat_sum, fc_w, fc_b)


# speedup vs baseline: 1.7469x; 1.7469x over previous
import functools
import jax
import jax.numpy as jnp
from jax import lax
from jax.experimental import pallas as pl
from jax.experimental.pallas import tpu as pltpu
from jax.experimental.pallas import tpu_sc as plsc

NC = 2
NS = 16
LANES = 16
NW = NC * NS
UCOLS = 4


def _sc_pack_tables(embT, c0T, c1T, c2T, emb_tail, c0_tail, c1_tail,
                    c2_tail):
  D, V = embT.shape
  CV = c0T.shape[1]
  emb_cols = V // 128
  cat_cols = CV // 128
  emb_units = emb_cols // UCOLS
  cat_units = cat_cols // UCOLS
  cat_rem = cat_cols - cat_units * UCOLS

  mesh = plsc.VectorSubcoreMesh(
      core_axis_name="c", subcore_axis_name="s", num_cores=NC,
      num_subcores=NS)

  @functools.partial(
      pl.kernel,
      compiler_params=pltpu.CompilerParams(
          use_tc_tiling_on_sc=True, needs_layout_passes=False),
      out_type=(
          jax.ShapeDtypeStruct((V * D,), jnp.float32),
          jax.ShapeDtypeStruct((CV * D,), jnp.float32),
          jax.ShapeDtypeStruct((CV * D,), jnp.float32),
          jax.ShapeDtypeStruct((CV * D,), jnp.float32),
      ),
      mesh=mesh,
      scratch_types=[
          pltpu.VMEM((32, 128 * UCOLS), jnp.float32),
          pltpu.VMEM((32, 128 * UCOLS), jnp.float32),
          pltpu.VMEM((4096 * UCOLS,), jnp.float32),
          pltpu.VMEM((4096 * UCOLS,), jnp.float32),
          pltpu.VMEM((2048,), jnp.float32),
          pltpu.SemaphoreType.DMA,
          pltpu.SemaphoreType.DMA,
          pltpu.SemaphoreType.DMA,
          pltpu.SemaphoreType.DMA,
      ],
  )
  def a_fn(embT_ref, c0T_ref, c1T_ref, c2T_ref, et_ref, ct0_ref, ct1_ref,
           ct2_ref, oe_ref, oc0_ref, oc1_ref, oc2_ref,
           tbuf0, tbuf1, pbuf0, pbuf1, tailb, sin0, sin1, sout0, sout1):
    wid = lax.axis_index("s") * NC + lax.axis_index("c")
    iota32 = lax.iota(jnp.int32, LANES) * 32
    tbufs = (tbuf0, tbuf1)
    pbufs = (pbuf0, pbuf1)
    sins = (sin0, sin1)
    souts = (sout0, sout1)
    UW = 4096 * UCOLS

    def pack_table(src_ref, out_ref, nunits):
      my_units = (nunits - wid + NW - 1) // NW

      def in_copies(u, b):
        c0 = (u * NW + wid) * UCOLS * 128
        return [
            pltpu.make_async_copy(
                src_ref.at[pl.ds(8 * dh, 8), pl.ds(c0, 128 * UCOLS)],
                tbufs[b].at[pl.ds(8 * dh, 8), :], sins[b])
            for dh in range(4)
        ]

      def out_copy(u, b):
        c0 = (u * NW + wid) * UCOLS * 128
        return pltpu.make_async_copy(
            pbufs[b], out_ref.at[pl.ds(c0 * 32, UW)], souts[b])

      def transpose(b):
        def d_body(d, dc):
          for k in range(UCOLS):
            for g in range(8):
              off = k * 128 + 16 * g
              v = tbufs[b][d, pl.ds(off, 16)]
              plsc.store_scatter(pbufs[b], [iota32 + (off * 32 + d)], v)
          return dc
        lax.fori_loop(0, 32, d_body, 0)

      @pl.when(my_units > 0)
      def _():
        for c in in_copies(0, 0):
          c.start()

      @pl.when(my_units > 1)
      def _():
        for c in in_copies(1, 1):
          c.start()

      def ring_body(n, carry):
        for b in range(2):
          u = n + b
          u_next = jnp.minimum(u + 2, my_units - 1)
          u_prev = jnp.maximum(u - 2, 0)

          @pl.when(u < my_units)
          def _(u=u, b=b, u_next=u_next, u_prev=u_prev):
            for c in in_copies(u, b):
              c.wait()

            @pl.when(u >= 2)
            def _():
              out_copy(u_prev, b).wait()
            transpose(b)
            out_copy(u, b).start()

            @pl.when(u + 2 < my_units)
            def _():
              for c in in_copies(u_next, b):
                c.start()
        return carry

      nhalf = (my_units + 1) // 2
      lax.fori_loop(0, nhalf, lambda h, c: ring_body(h * 2, c), 0)

      for b in range(2):
        @pl.when(my_units > b)
        def _(b=b):
          last = jnp.maximum(((my_units - 1 - b) // 2) * 2 + b, 0)
          out_copy(last, b).wait()

    pack_table(embT_ref, oe_ref, emb_units)
    pack_table(c0T_ref, oc0_ref, cat_units)
    pack_table(c1T_ref, oc1_ref, cat_units)
    pack_table(c2T_ref, oc2_ref, cat_units)

    if cat_rem:
      for i, (s_ref, o_ref) in enumerate(
          ((c0T_ref, oc0_ref), (c1T_ref, oc1_ref), (c2T_ref, oc2_ref))):
        @pl.when(wid == 4 + i)
        def _(s_ref=s_ref, o_ref=o_ref):
          c0 = (cat_cols - 1) * 128
          for dh in range(4):
            pltpu.sync_copy(s_ref.at[pl.ds(8 * dh, 8), pl.ds(c0, 128)],
                            tbuf0.at[pl.ds(8 * dh, 8), pl.ds(0, 128)])

          def d_body(d, dc):
            for g in range(8):
              off = 16 * g
              v = tbuf0[d, pl.ds(off, 16)]
              plsc.store_scatter(pbuf0, [iota32 + (off * 32 + d)], v)
            return dc
          lax.fori_loop(0, 32, d_body, 0)
          pltpu.sync_copy(pbuf0.at[pl.ds(0, 4096)],
                          o_ref.at[pl.ds(c0 * 32, 4096)])

    @pl.when(wid == 0)
    def _():
      pltpu.sync_copy(et_ref, tailb)
      pltpu.sync_copy(tailb, oe_ref.at[pl.ds(emb_cols * 4096, 2048)])

    for i, (t_ref, o_ref) in enumerate(
        ((ct0_ref, oc0_ref), (ct1_ref, oc1_ref), (ct2_ref, oc2_ref))):
      @pl.when(wid == i + 1)
      def _(t_ref=t_ref, o_ref=o_ref):
        pltpu.sync_copy(t_ref, tailb.at[pl.ds(0, 1024)])
        pltpu.sync_copy(tailb.at[pl.ds(0, 1024)],
                        o_ref.at[pl.ds(cat_cols * 4096, 1024)])

  return a_fn(embT, c0T, c1T, c2T, emb_tail, c0_tail, c1_tail, c2_tail)


def _sc_gather_pool(enc, add0, add1, add2, emb, cat0, cat1, cat2):
  B, L = enc.shape
  V, D = emb.shape
  rows_per_w = B // NW
  groups = rows_per_w // LANES

  mesh = plsc.VectorSubcoreMesh(
      core_axis_name="c", subcore_axis_name="s", num_cores=NC,
      num_subcores=NS)

  @functools.partial(
      pl.kernel,
      compiler_params=pltpu.CompilerParams(
          use_tc_tiling_on_sc=False, needs_layout_passes=False),
      out_type=(
          jax.ShapeDtypeStruct((B, D), jnp.float32),
          jax.ShapeDtypeStruct((B,), jnp.float32),
          jax.ShapeDtypeStruct((3, B, D), jnp.float32)
      ),
      mesh=mesh,
      scratch_types=[
          pltpu.VMEM((LANES, L), jnp.int32),
          pltpu.VMEM((LANES * L, D), jnp.float32),
          pltpu.VMEM((3, LANES), jnp.int32),
          pltpu.VMEM((3, LANES, D), jnp.float32),
          pltpu.VMEM((D, LANES), jnp.float32),
          pltpu.VMEM((LANES,), jnp.float32),
          pltpu.VMEM((LANES, D), jnp.float32),
          pltpu.SemaphoreType.DMA,
          pltpu.SemaphoreType.DMA,
      ],
  )
  def b_fn(enc_ref, a0_ref, a1_ref, a2_ref, emb_ref, c0_ref, c1_ref, c2_ref,
           pooled_out, cnt_out, cat_out,
           idx_v, rows_v, cidxT_v, crows_v, pooled_v, cnt_v,
           outp_v, sem, sem2):
    wid = lax.axis_index("s") * NC + lax.axis_index("c")
    lanes_iota = lax.iota(jnp.int32, LANES)
    zeros16 = jnp.zeros((LANES,), jnp.float32)
    base_rows = lanes_iota * L
    cat_refs = (c0_ref, c1_ref, c2_ref)

    def group_body(g, carry):
      b0 = (wid * groups + g) * LANES
      pltpu.sync_copy(enc_ref.at[pl.ds(b0, LANES), :], idx_v)
      for i, a_ref in enumerate((a0_ref, a1_ref, a2_ref)):
        pltpu.sync_copy(a_ref.at[pl.ds(b0, LANES)], cidxT_v.at[i])
      copies = [
          pltpu.async_copy(emb_ref.at[idx_v.at[j]],
                           rows_v.at[pl.ds(j * L, L), :], sem)
          for j in range(LANES)
      ]
      cat_copies = [
          pltpu.async_copy(cat_refs[i].at[cidxT_v.at[i]], crows_v.at[i],
                           sem2)
          for i in range(3)
      ]
      for d in range(D):
        pooled_v[d] = zeros16
      cnt_v[...] = zeros16
      for c in copies:
        c.wait()

      def token_body(t, tc):
        ridx = base_rows + t
        s0 = s1 = s2 = s3 = zeros16
        for d in range(D):
          v = plsc.load_gather(
              rows_v, [ridx, jnp.full((LANES,), d, jnp.int32)])
          plsc.addupdate(pooled_v.at[d], v)
          if d % 4 == 0:
            s0 = s0 + v
          elif d % 4 == 1:
            s1 = s1 + v
          elif d % 4 == 2:
            s2 = s2 + v
          else:
            s3 = s3 + v
        s = (s0 + s1) + (s2 + s3)
        plsc.addupdate(
            cnt_v.at[:],
            jnp.where(s != 0.0, jnp.float32(1.0), jnp.float32(0.0)))
        return tc

      lax.fori_loop(0, L, token_body, 0)

      for d in range(D):
        plsc.store_scatter(
            outp_v, [lanes_iota, jnp.full((LANES,), d, jnp.int32)],
            pooled_v[d])
      pltpu.sync_copy(outp_v, pooled_out.at[pl.ds(b0, LANES), :])
      pltpu.sync_copy(cnt_v, cnt_out.at[pl.ds(b0, LANES)])
      for c in cat_copies:
        c.wait()
      for i in range(3):
        pltpu.sync_copy(crows_v.at[i], cat_out.at[i, pl.ds(b0, LANES), :])
      return carry

    lax.fori_loop(0, groups, group_body, 0)

  return b_fn(enc, add0, add1, add2, emb, cat0, cat1, cat2)


def _tc_finalize_matmul(pooled, cnt, cats, fc_w, fc_b):
  B, D = pooled.shape
  C = fc_w.shape[1]
  BB = 512
  cnt2 = cnt.reshape(B, 1)
  fb2 = fc_b.reshape(1, C)

  def tc_body(p_ref, c_ref, cat_ref, w_ref, b_ref, o_ref):
    q = p_ref[...] / c_ref[...]
    q = jnp.where(q != q, jnp.float32(0.0), q)
    big = jnp.float32(3.4028234663852886e38)
    q = jnp.where(q == jnp.inf, big, q)
    q = jnp.where(q == -jnp.inf, -big, q)
    x = q + cat_ref[0] + cat_ref[1] + cat_ref[2]
    o_ref[...] = (
        jnp.dot(x, w_ref[...], preferred_element_type=jnp.float32)
        + b_ref[...])

  return pl.pallas_call(
      tc_body,
      grid=(B // BB,),
      in_specs=[
          pl.BlockSpec((BB, D), lambda i: (i, 0)),
          pl.BlockSpec((BB, 1), lambda i: (i, 0)),
          pl.BlockSpec((3, BB, D), lambda i: (0, i, 0)),
          pl.BlockSpec((D, C), lambda i: (0, 0)),
          pl.BlockSpec((1, C), lambda i: (0, 0)),
      ],
      out_specs=pl.BlockSpec((BB, C), lambda i: (i, 0)),
      out_shape=jax.ShapeDtypeStruct((B, C), jnp.float32),
  )(pooled, cnt2, cats, fc_w, fb2)


def kernel(encoded_text, additional_inputs, emb_table, cat_emb0, cat_emb1,
           cat_emb2, fc_w, fc_b):
  B, L = encoded_text.shape
  V, D = emb_table.shape
  CV = cat_emb0.shape[0]
  emb_main = (V // 128) * 128
  cat_main = (CV // 128) * 128

  enc = encoded_text.astype(jnp.int32)
  add_i = additional_inputs.astype(jnp.int32)

  emb_tail = emb_table[emb_main:, :].reshape(-1)
  c_tails = [t[cat_main:, :].reshape(-1)
             for t in (cat_emb0, cat_emb1, cat_emb2)]

  oe, oc0, oc1, oc2 = _sc_pack_tables(
      emb_table.T, cat_emb0.T, cat_emb1.T, cat_emb2.T,
      emb_tail, *c_tails)

  pooled, cnt, cats = _sc_gather_pool(
      enc, add_i[:, 0], add_i[:, 1], add_i[:, 2],
      oe.reshape(V, D), oc0.reshape(CV, D), oc1.reshape(CV, D),
      oc2.reshape(CV, D))
  return _tc_finalize_matmul(pooled, cnt, cats, fc_w, fc_b)
